# baseline (speedup 1.0000x reference)
import functools

import jax
import jax.numpy as jnp
from jax import lax
from jax.experimental import pallas as pl
from jax.experimental.pallas import tpu as pltpu
from jax.experimental.pallas import tpu_sc as plsc

B, N, M, K, C_IN, C_OUT = 16, 4096, 1024, 32, 128, 256


_MLP_R = 2048


def _mlp_body(x_ref, w1_ref, b1_ref, w2_ref, b2_ref, out_ref):
    x = x_ref[...]
    h = jnp.dot(x, w1_ref[...], preferred_element_type=jnp.float32)
    h = jnp.maximum(h + b1_ref[...], 0.0)
    y = jnp.dot(h, w2_ref[...], preferred_element_type=jnp.float32)
    out_ref[...] = jnp.maximum(y + b2_ref[...], 0.0)


def _mlp(feats2d, W1, b1, W2, b2):
    rows = feats2d.shape[0]
    return pl.pallas_call(
        _mlp_body,
        grid=(rows // _MLP_R,),
        in_specs=[
            pl.BlockSpec((_MLP_R, C_IN), lambda i: (i, 0)),
            pl.BlockSpec((C_IN, C_OUT), lambda i: (0, 0)),
            pl.BlockSpec((1, C_OUT), lambda i: (0, 0)),
            pl.BlockSpec((C_OUT, C_OUT), lambda i: (0, 0)),
            pl.BlockSpec((1, C_OUT), lambda i: (0, 0)),
        ],
        out_specs=pl.BlockSpec((_MLP_R, C_OUT), lambda i: (i, 0)),
        out_shape=jax.ShapeDtypeStruct((rows, C_OUT), jnp.float32),
    )(feats2d, W1, b1, W2, b2)


_KNN_R = 512


def _knn_body(posT_ref, idx_ref, newpos_ref, gidx_ref, *, b0):
    b = pl.program_id(0) + b0
    pT = posT_ref[0]
    idxr = idx_ref[0, 0, 0]

    ioti = lax.broadcasted_iota(jnp.int32, (1, N), 1)
    ehb = idxr[:, None] == ioti
    ninf = jnp.float32(-jnp.inf)
    qj = [jnp.max(jnp.where(ehb, pT[j:j + 1, :], ninf), axis=1, keepdims=True)
          for j in range(3)]
    newpos_ref[0] = jnp.concatenate(qj, axis=1)

    d = jnp.zeros((_KNN_R, N), jnp.float32)
    for j in range(3):
        d = d + (qj[j] - pT[j:j + 1, :]) ** 2

    inf = jnp.float32(jnp.inf)
    base = b * N
    m = jnp.min(d, axis=1, keepdims=True)
    for k in range(K):
        am = jnp.min(jnp.where(d == m, ioti, N), axis=1, keepdims=True)
        gidx_ref[0, :, pl.ds(k, 1)] = am + base
        if k < K - 1:
            m = jnp.min(jnp.where(d > m, d, inf), axis=1, keepdims=True)


def _knn(posT, idx4, b0):
    nb = posT.shape[0]
    return pl.pallas_call(
        functools.partial(_knn_body, b0=b0),
        grid=(nb, M // _KNN_R),
        in_specs=[
            pl.BlockSpec((1, 3, N), lambda b, i: (b, 0, 0)),
            pl.BlockSpec((1, 1, 1, _KNN_R), lambda b, i: (b, i, 0, 0)),
        ],
        out_specs=[
            pl.BlockSpec((1, _KNN_R, 3), lambda b, i: (b, i, 0)),
            pl.BlockSpec((1, _KNN_R, K), lambda b, i: (b, i, 0)),
        ],
        out_shape=[
            jax.ShapeDtypeStruct((nb, M, 3), jnp.float32),
            jax.ShapeDtypeStruct((nb, M, K), jnp.int32),
        ],
    )(posT, idx4)


_NW = 32
_GRP = 4
_GR = _GRP * K


def _scmax(h2d, gidx_flat):
    ncent = gidx_flat.shape[0] // K
    cent_w = ncent // _NW
    rows_w = cent_w * K
    ngrp = cent_w // _GRP
    mesh = plsc.VectorSubcoreMesh(core_axis_name="c", subcore_axis_name="s")

    @functools.partial(
        pl.kernel,
        mesh=mesh,
        out_type=jax.ShapeDtypeStruct((ncent, C_OUT), jnp.float32),
        scratch_types=[
            pltpu.VMEM((rows_w,), jnp.int32),
            pltpu.VMEM((_GR, C_OUT), jnp.float32),
            pltpu.VMEM((_GR, C_OUT), jnp.float32),
            pltpu.VMEM((_GRP, C_OUT), jnp.float32),
            pltpu.SemaphoreType.DMA,
            pltpu.SemaphoreType.DMA,
        ],
    )
    def kern(h_hbm, gidx_hbm, out_hbm, idx_v, buf0, buf1, ostage, sem0, sem1):
        wid = lax.axis_index("s") * 2 + lax.axis_index("c")
        ibase = wid * rows_w
        cbase = wid * cent_w
        pltpu.sync_copy(gidx_hbm.at[pl.ds(ibase, rows_w)], idx_v)

        def start(g, buf, sem):
            pltpu.async_copy(h_hbm.at[idx_v.at[pl.ds(g * _GR, _GR)]], buf, sem)

        def wait(buf, sem):
            pltpu.make_async_copy(h_hbm.at[idx_v.at[pl.ds(0, _GR)]], buf, sem).wait()

        def compute_out(g, buf):
            for t in range(_GRP):
                def cbody(c, _):
                    col = c * 16
                    acc = buf[t * K, pl.ds(col, 16)]
                    for r in range(1, K):
                        acc = jnp.maximum(acc, buf[t * K + r, pl.ds(col, 16)])
                    ostage[t, pl.ds(col, 16)] = acc
                    return 0
                lax.fori_loop(0, C_OUT // 16, cbody, 0)
            pltpu.sync_copy(ostage, out_hbm.at[pl.ds(cbase + g * _GRP, _GRP)])

        start(0, buf0, sem0)
        start(1, buf1, sem1)

        def pair(g2, _):
            g0 = g2 * 2
            g1 = g0 + 1
            wait(buf0, sem0)
            compute_out(g0, buf0)

            @pl.when(g0 + 2 < ngrp)
            def _():
                start(g0 + 2, buf0, sem0)

            wait(buf1, sem1)
            compute_out(g1, buf1)

            @pl.when(g1 + 2 < ngrp)
            def _():
                start(g1 + 2, buf1, sem1)

            return 0

        lax.fori_loop(0, ngrp // 2, pair, 0)

    return kern(h2d, gidx_flat)


def kernel(pos, features, W1, b1, W2, b2, idx):
    idx4 = idx.astype(jnp.int32).reshape(B, M // _KNN_R, 1, _KNN_R)
    posT = jnp.swapaxes(pos, 1, 2)
    h2d = _mlp(features.reshape(B * N, C_IN), W1,
               b1.reshape(1, C_OUT), W2, b2.reshape(1, C_OUT))
    hb = B // 4
    nps, outs = [], []
    for s in range(4):
        nps_i, g_i = _knn(posT[s * hb:(s + 1) * hb],
                          idx4[s * hb:(s + 1) * hb], s * hb)
        o_i = _scmax(h2d, g_i.reshape(hb * M * K))
        nps.append(nps_i)
        outs.append(o_i.reshape(hb, M, C_OUT))
    return jnp.concatenate(nps, axis=0), jnp.concatenate(outs, axis=0)


# speedup vs baseline: 1.1375x; 1.1375x over previous
import functools

import jax
import jax.numpy as jnp
from jax import lax
from jax.experimental import pallas as pl
from jax.experimental.pallas import tpu as pltpu
from jax.experimental.pallas import tpu_sc as plsc

B, N, M, K, C_IN, C_OUT = 16, 4096, 1024, 32, 128, 256


_MLP_R = 2048


def _mlp_body(x_ref, w1_ref, b1_ref, w2_ref, b2_ref, out_ref):
    x = x_ref[...]
    h = jnp.dot(x, w1_ref[...], preferred_element_type=jnp.float32)
    h = jnp.maximum(h + b1_ref[...], 0.0)
    y = jnp.dot(h, w2_ref[...], preferred_element_type=jnp.float32)
    out_ref[...] = jnp.maximum(y + b2_ref[...], 0.0)


def _mlp(feats2d, W1, b1, W2, b2):
    rows = feats2d.shape[0]
    return pl.pallas_call(
        _mlp_body,
        grid=(rows // _MLP_R,),
        in_specs=[
            pl.BlockSpec((_MLP_R, C_IN), lambda i: (i, 0)),
            pl.BlockSpec((C_IN, C_OUT), lambda i: (0, 0)),
            pl.BlockSpec((1, C_OUT), lambda i: (0, 0)),
            pl.BlockSpec((C_OUT, C_OUT), lambda i: (0, 0)),
            pl.BlockSpec((1, C_OUT), lambda i: (0, 0)),
        ],
        out_specs=pl.BlockSpec((_MLP_R, C_OUT), lambda i: (i, 0)),
        out_shape=jax.ShapeDtypeStruct((rows, C_OUT), jnp.float32),
    )(feats2d, W1, b1, W2, b2)


_KNN_R = 128


def _knn_body(posT_ref, idx_ref, newpos_ref, gidx_ref, *, b0):
    b = pl.program_id(0) + b0
    pT = posT_ref[0]
    idxr = idx_ref[0, 0, 0]

    ioti = lax.broadcasted_iota(jnp.int32, (1, N), 1)
    ehb = idxr[:, None] == ioti
    ninf = jnp.float32(-jnp.inf)
    qj = [jnp.max(jnp.where(ehb, pT[j:j + 1, :], ninf), axis=1, keepdims=True)
          for j in range(3)]
    newpos_ref[0] = jnp.concatenate(qj, axis=1)

    d = jnp.zeros((_KNN_R, N), jnp.float32)
    for j in range(3):
        d = d + (qj[j] - pT[j:j + 1, :]) ** 2

    inf = jnp.float32(jnp.inf)
    base = b * N
    m = jnp.min(d, axis=1, keepdims=True)
    for k in range(K):
        am = jnp.min(jnp.where(d == m, ioti, N), axis=1, keepdims=True)
        gidx_ref[0, :, pl.ds(k, 1)] = am + base
        if k < K - 1:
            m = jnp.min(jnp.where(d > m, d, inf), axis=1, keepdims=True)


def _knn(posT, idx4, b0):
    nb = posT.shape[0]
    return pl.pallas_call(
        functools.partial(_knn_body, b0=b0),
        grid=(nb, M // _KNN_R),
        in_specs=[
            pl.BlockSpec((1, 3, N), lambda b, i: (b, 0, 0)),
            pl.BlockSpec((1, 1, 1, _KNN_R), lambda b, i: (b, i, 0, 0)),
        ],
        out_specs=[
            pl.BlockSpec((1, _KNN_R, 3), lambda b, i: (b, i, 0)),
            pl.BlockSpec((1, _KNN_R, K), lambda b, i: (b, i, 0)),
        ],
        out_shape=[
            jax.ShapeDtypeStruct((nb, M, 3), jnp.float32),
            jax.ShapeDtypeStruct((nb, M, K), jnp.int32),
        ],
    )(posT, idx4)


_NW = 32
_GRP = 4
_GR = _GRP * K


def _scmax(h2d, gidx_flat):
    ncent = gidx_flat.shape[0] // K
    cent_w = ncent // _NW
    rows_w = cent_w * K
    ngrp = cent_w // _GRP
    mesh = plsc.VectorSubcoreMesh(core_axis_name="c", subcore_axis_name="s")

    @functools.partial(
        pl.kernel,
        mesh=mesh,
        out_type=jax.ShapeDtypeStruct((ncent, C_OUT), jnp.float32),
        scratch_types=[
            pltpu.VMEM((rows_w,), jnp.int32),
            pltpu.VMEM((_GR, C_OUT), jnp.float32),
            pltpu.VMEM((_GR, C_OUT), jnp.float32),
            pltpu.VMEM((_GRP, C_OUT), jnp.float32),
            pltpu.SemaphoreType.DMA,
            pltpu.SemaphoreType.DMA,
        ],
    )
    def kern(h_hbm, gidx_hbm, out_hbm, idx_v, buf0, buf1, ostage, sem0, sem1):
        wid = lax.axis_index("s") * 2 + lax.axis_index("c")
        ibase = wid * rows_w
        cbase = wid * cent_w
        pltpu.sync_copy(gidx_hbm.at[pl.ds(ibase, rows_w)], idx_v)

        def start(g, buf, sem):
            pltpu.async_copy(h_hbm.at[idx_v.at[pl.ds(g * _GR, _GR)]], buf, sem)

        def wait(buf, sem):
            pltpu.make_async_copy(h_hbm.at[idx_v.at[pl.ds(0, _GR)]], buf, sem).wait()

        def compute_out(g, buf):
            for t in range(_GRP):
                def cbody(c, _):
                    col = c * 16
                    acc = buf[t * K, pl.ds(col, 16)]
                    for r in range(1, K):
                        acc = jnp.maximum(acc, buf[t * K + r, pl.ds(col, 16)])
                    ostage[t, pl.ds(col, 16)] = acc
                    return 0
                lax.fori_loop(0, C_OUT // 16, cbody, 0)
            pltpu.sync_copy(ostage, out_hbm.at[pl.ds(cbase + g * _GRP, _GRP)])

        start(0, buf0, sem0)
        start(1, buf1, sem1)

        def pair(g2, _):
            g0 = g2 * 2
            g1 = g0 + 1
            wait(buf0, sem0)
            compute_out(g0, buf0)

            @pl.when(g0 + 2 < ngrp)
            def _():
                start(g0 + 2, buf0, sem0)

            wait(buf1, sem1)
            compute_out(g1, buf1)

            @pl.when(g1 + 2 < ngrp)
            def _():
                start(g1 + 2, buf1, sem1)

            return 0

        lax.fori_loop(0, ngrp // 2, pair, 0)

    return kern(h2d, gidx_flat)


def kernel(pos, features, W1, b1, W2, b2, idx):
    idx4 = idx.astype(jnp.int32).reshape(B, M // _KNN_R, 1, _KNN_R)
    posT = jnp.swapaxes(pos, 1, 2)
    h2d = _mlp(features.reshape(B * N, C_IN), W1,
               b1.reshape(1, C_OUT), W2, b2.reshape(1, C_OUT))
    hb = B // 4
    nps, outs = [], []
    for s in range(4):
        nps_i, g_i = _knn(posT[s * hb:(s + 1) * hb],
                          idx4[s * hb:(s + 1) * hb], s * hb)
        o_i = _scmax(h2d, g_i.reshape(hb * M * K))
        nps.append(nps_i)
        outs.append(o_i.reshape(hb, M, C_OUT))
    return jnp.concatenate(nps, axis=0), jnp.concatenate(outs, axis=0)
